# Initial kernel scaffold; baseline (speedup 1.0000x reference)
import jax
import jax.numpy as jnp
from jax.experimental import pallas as pl


def kernel(x, w1m, b1, w2r, b2, fc1_wt, fc1_b, fc2_wt, fc2_b):
    raise NotImplementedError("write your pallas kernel here")


# speedup vs baseline: 16.6914x; 16.6914x over previous
import functools

import numpy as np
import jax
import jax.numpy as jnp
from jax.experimental import pallas as pl
from jax.experimental.pallas import tpu as pltpu

NB = 256
ROWPAD = 1024


def _build_static_maps():
    rho = np.arange(128)[:, None] // 32
    v = np.arange(128)[:, None] % 32
    gw = np.arange(60)[None, :]
    dh = gw // 30
    dw = (gw // 15) % 2
    w15 = gw % 15
    i1 = rho - dh
    j1 = v - 2 * w15 - dw
    m1 = (i1 >= 0) & (i1 < 3) & (j1 >= 0) & (j1 < 3)
    t_idx = np.where(m1, i1 * 3 + j1, 0).astype(np.int32)

    kappa = np.arange(8192)[:, None]
    rho2 = kappa // ROWPAD
    wc = (kappa % ROWPAD) // 64
    c = kappa % 64
    pg = np.arange(16)[None, :]
    par = pg // 8
    ow = pg % 8
    i2 = rho2 - par
    j2 = wc - ow
    m2 = (i2 >= 0) & (i2 < 7) & (j2 >= 0) & (j2 < 7) & (c < 64)
    w_idx = np.where(m2, i2 * 448 + j2 * 64 + c, 0).astype(np.int32)

    l = np.arange(256)
    perm = (l % 16) * 16 + (l // 64) * 4 + ((l % 64) // 16)
    return t_idx, m1, w_idx, m2, perm.astype(np.int32)


_T_IDX, _T_MASK, _W_IDX, _W_MASK, _FC1_PERM = _build_static_maps()


def _lenet_kernel(x_ref, t_ref, w2_ref, b1_ref, b2_ref, f1w_ref, f1b_ref,
                  f2w_ref, f2b_ref, o_ref, y1s):
    xb = x_ref[...].astype(jnp.bfloat16)
    t = t_ref[...]
    zeros64 = jnp.zeros((NB, ROWPAD - 960), jnp.bfloat16)
    b1 = b1_ref[...]
    for r in range(15):
        z = jnp.dot(xb[:, 64 * r:64 * r + 128], t,
                    preferred_element_type=jnp.float32)
        m = jnp.maximum(jnp.maximum(z[:, :960], z[:, 960:1920]),
                        jnp.maximum(z[:, 1920:2880], z[:, 2880:3840]))
        y = jnp.tanh(m + b1)
        y1s[:, ROWPAD * r:ROWPAD * r + 960] = y.astype(jnp.bfloat16)
        y1s[:, ROWPAD * r + 960:ROWPAD * (r + 1)] = zeros64

    w2 = w2_ref[...]
    feats = []
    for p in range(4):
        zp = jnp.dot(y1s[:, 2 * ROWPAD * p:2 * ROWPAD * p + 8192], w2,
                     preferred_element_type=jnp.float32)
        vp = jnp.maximum(zp[:, :128], zp[:, 128:])
        feats.extend(
            jnp.maximum(vp[:, 32 * q:32 * q + 16], vp[:, 32 * q + 16:32 * q + 32])
            for q in range(4))
    feat = jnp.tanh(jnp.concatenate(feats, axis=1) + b2_ref[...])

    h = jnp.tanh(
        jnp.dot(feat.astype(jnp.bfloat16), f1w_ref[...],
                preferred_element_type=jnp.float32) + f1b_ref[...])
    z2 = jnp.dot(h.astype(jnp.bfloat16), f2w_ref[...],
                 preferred_element_type=jnp.float32) + f2b_ref[...]
    mx = jnp.max(z2, axis=1, keepdims=True)
    s = jnp.sum(jnp.exp(z2 - mx), axis=1, keepdims=True)
    o_ref[...] = z2 - mx - jnp.log(s)


@jax.jit
def _forward(x, w1m, b1, w2r, b2, fc1_wt, fc1_b, fc2_wt, fc2_b):
    x2d = x.reshape(-1, 1024).astype(jnp.float32)
    B = x2d.shape[0]
    Bp = (B + NB - 1) // NB * NB
    if Bp != B:
        x2d = jnp.pad(x2d, ((0, Bp - B), (0, 0)))

    t_mat = jnp.where(_T_MASK[:, :, None], w1m[_T_IDX], 0.0)
    t_mat = t_mat.reshape(128, 3840).astype(jnp.bfloat16)
    w2f = w2r.reshape(3136, 16)
    w2t = jnp.where(_W_MASK[:, :, None], w2f[_W_IDX], 0.0)
    w2t = w2t.reshape(8192, 256).astype(jnp.bfloat16)
    b1t = jnp.tile(b1.reshape(1, 64), (1, 15))
    b2t = jnp.tile(b2.reshape(1, 16), (1, 16))
    f1p = fc1_wt[_FC1_PERM].astype(jnp.bfloat16)
    f1b = fc1_b.reshape(1, 200)
    f2w = fc2_wt.astype(jnp.bfloat16)
    f2b = fc2_b.reshape(1, 10)

    out = pl.pallas_call(
        _lenet_kernel,
        out_shape=jax.ShapeDtypeStruct((Bp, 10), jnp.float32),
        grid=(Bp // NB,),
        in_specs=[
            pl.BlockSpec((NB, 1024), lambda b: (b, 0)),
            pl.BlockSpec((128, 3840), lambda b: (0, 0)),
            pl.BlockSpec((8192, 256), lambda b: (0, 0)),
            pl.BlockSpec((1, 960), lambda b: (0, 0)),
            pl.BlockSpec((1, 256), lambda b: (0, 0)),
            pl.BlockSpec((256, 200), lambda b: (0, 0)),
            pl.BlockSpec((1, 200), lambda b: (0, 0)),
            pl.BlockSpec((200, 10), lambda b: (0, 0)),
            pl.BlockSpec((1, 10), lambda b: (0, 0)),
        ],
        out_specs=pl.BlockSpec((NB, 10), lambda b: (b, 0)),
        scratch_shapes=[pltpu.VMEM((NB, 15 * ROWPAD), jnp.bfloat16)],
        compiler_params=pltpu.CompilerParams(
            dimension_semantics=("parallel",),
            vmem_limit_bytes=100 * 1024 * 1024),
    )(x2d, t_mat, w2t, b1t, b2t, f1p, f1b, f2w, f2b)
    return out[:B]


def kernel(x, w1m, b1, w2r, b2, fc1_wt, fc1_b, fc2_wt, fc2_b):
    return _forward(x, w1m, b1, w2r, b2, fc1_wt, fc1_b, fc2_wt, fc2_b)
